# baseline (speedup 1.0000x reference)
import functools

import jax
import jax.numpy as jnp
from jax.experimental import pallas as pl
from jax.experimental.pallas import tpu as pltpu

_CHUNKS = 4


def _gcn_body(y_hbm, w1_ref, b1_ref, w2_ref, b2_ref, m1_ref, c1_ref,
              m2_ref, c2_ref, out_ref, y_vmem, a_bf, sems, *, n_batch, n):
    nc = _CHUNKS
    rows = n // nc
    copies = []
    for b in range(n_batch):
        for c in range(nc):
            cp = pltpu.make_async_copy(
                y_hbm.at[b, c * rows:(c + 1) * rows, :],
                y_vmem.at[b, c * rows:(c + 1) * rows, :],
                sems.at[b * nc + c])
            cp.start()
            copies.append(cp)

    tile = 128
    row_i = jax.lax.broadcasted_iota(jnp.int32, (tile, tile), 0)
    col_i = jax.lax.broadcasted_iota(jnp.int32, (tile, tile), 1)
    mask = row_i == col_i

    r_rows = []
    for b in range(n_batch):
        csum = None
        for c in range(nc):
            copies[b * nc + c].wait()
            sl = slice(c * rows, (c + 1) * rows)
            v = y_vmem[b, sl, :]
            part = jnp.sum(v, axis=0, keepdims=True)
            csum = part if csum is None else csum + part
            a_bf[b, sl, :] = v.astype(jnp.bfloat16)

        diag = jnp.concatenate(
            [jnp.sum(jnp.where(mask,
                               y_vmem[b, t * tile:(t + 1) * tile,
                                      t * tile:(t + 1) * tile], 0.0),
                     axis=0, keepdims=True)
             for t in range(n // tile)], axis=1)
        loop_w = jnp.where(diag == 0.0, 1.0, 0.0)
        deg = csum + loop_w
        dinv = jnp.where(deg > 0.0,
                         jax.lax.rsqrt(jnp.where(deg > 0.0, deg, 1.0)), 0.0)

        abf = a_bf[b]

        t1 = jnp.dot(dinv.astype(jnp.bfloat16), abf,
                     preferred_element_type=jnp.float32)
        s = dinv * (t1 + dinv * loop_w)
        x1t = jnp.maximum(w1_ref[...] * s + b1_ref[...], 0.0)

        h2t = jnp.dot(w2_ref[...], x1t,
                      preferred_element_type=jnp.float32)
        gt = dinv * h2t
        zt = jnp.dot(gt.astype(jnp.bfloat16), abf,
                     preferred_element_type=jnp.float32)
        out2t = dinv * (zt + loop_w * gt) + b2_ref[...]
        r_rows.append(jnp.max(out2t, axis=0, keepdims=True))

    rr = jnp.concatenate(r_rows, axis=0)
    o1 = jax.lax.dot_general(
        rr, m1_ref[...], (((1,), (1,)), ((), ())),
        preferred_element_type=jnp.float32) + c1_ref[...]
    o2 = jax.lax.dot_general(
        o1, m2_ref[...], (((1,), (1,)), ((), ())),
        preferred_element_type=jnp.float32) + c2_ref[...]
    out_ref[...] = o2


def kernel(y, W1, b1, W2, b2, M1, c1, M2, c2):
    B, N = y.shape[0], y.shape[1]
    H = W1.shape[0]
    w1c = W1.reshape(H, 1)
    b1c = b1.reshape(H, 1)
    b2c = b2.reshape(-1, 1)
    c1r = c1.reshape(1, -1)
    c2r = c2.reshape(1, -1)

    vmem = pl.BlockSpec(memory_space=pltpu.MemorySpace.VMEM)
    return pl.pallas_call(
        functools.partial(_gcn_body, n_batch=B, n=N),
        in_specs=[
            pl.BlockSpec(memory_space=pl.ANY),
            vmem, vmem, vmem, vmem, vmem, vmem, vmem, vmem,
        ],
        out_specs=vmem,
        out_shape=jax.ShapeDtypeStruct((B, c2r.shape[1]), jnp.float32),
        scratch_shapes=[
            pltpu.VMEM((B, N, N), jnp.float32),
            pltpu.VMEM((B, N, N), jnp.bfloat16),
            pltpu.SemaphoreType.DMA((B * _CHUNKS,)),
        ],
    )(y, w1c, b1c, W2, b2c, M1, c1r, M2, c2r)


# speedup vs baseline: 1.0920x; 1.0920x over previous
import functools

import jax
import jax.numpy as jnp
from jax.experimental import pallas as pl
from jax.experimental.pallas import tpu as pltpu


def _gcn_body(y_ref, w1_ref, b1_ref, w2_ref, b2_ref, m1_ref, c1_ref,
              m2_ref, c2_ref, out_ref, r_scr, *, n_batch):
    b = pl.program_id(0)
    a = y_ref[0]
    n = a.shape[0]

    tile = 128
    row_i = jax.lax.broadcasted_iota(jnp.int32, (tile, tile), 0)
    col_i = jax.lax.broadcasted_iota(jnp.int32, (tile, tile), 1)
    mask = row_i == col_i
    diag = jnp.concatenate(
        [jnp.sum(jnp.where(mask,
                           y_ref[0, t * tile:(t + 1) * tile,
                                 t * tile:(t + 1) * tile], 0.0),
                 axis=0, keepdims=True)
         for t in range(n // tile)], axis=1)
    loop_w = jnp.where(diag == 0.0, 1.0, 0.0)
    deg = jnp.sum(a, axis=0, keepdims=True) + loop_w
    dinv = jnp.where(deg > 0.0, jax.lax.rsqrt(jnp.where(deg > 0.0, deg, 1.0)),
                     0.0)

    a_bf = a.astype(jnp.bfloat16)

    t1 = jnp.dot(dinv.astype(jnp.bfloat16), a_bf,
                 preferred_element_type=jnp.float32)
    s = dinv * (t1 + dinv * loop_w)
    x1t = jnp.maximum(w1_ref[...] * s + b1_ref[...], 0.0)

    h2t = jnp.dot(w2_ref[...], x1t,
                  preferred_element_type=jnp.float32)
    gt = dinv * h2t
    zt = jnp.dot(gt.astype(jnp.bfloat16), a_bf,
                 preferred_element_type=jnp.float32)
    out2t = dinv * (zt + loop_w * gt) + b2_ref[...]
    r_scr[pl.ds(b, 1), :] = jnp.max(out2t, axis=0, keepdims=True)

    @pl.when(b == n_batch - 1)
    def _():
        rr = r_scr[...]
        o1 = jax.lax.dot_general(
            rr, m1_ref[...], (((1,), (1,)), ((), ())),
            preferred_element_type=jnp.float32) + c1_ref[...]
        o2 = jax.lax.dot_general(
            o1, m2_ref[...], (((1,), (1,)), ((), ())),
            preferred_element_type=jnp.float32) + c2_ref[...]
        out_ref[...] = o2


def kernel(y, W1, b1, W2, b2, M1, c1, M2, c2):
    B, N = y.shape[0], y.shape[1]
    H = W1.shape[0]
    w1c = W1.reshape(H, 1)
    b1c = b1.reshape(H, 1)
    b2c = b2.reshape(-1, 1)
    c1r = c1.reshape(1, -1)
    c2r = c2.reshape(1, -1)

    vmem = pl.BlockSpec(memory_space=pltpu.MemorySpace.VMEM)
    return pl.pallas_call(
        functools.partial(_gcn_body, n_batch=B),
        grid=(B,),
        in_specs=[
            pl.BlockSpec((1, N, N), lambda b: (b, 0, 0)),
            vmem, vmem, vmem, vmem, vmem, vmem, vmem, vmem,
        ],
        out_specs=pl.BlockSpec((B, c2r.shape[1]), lambda b: (0, 0)),
        out_shape=jax.ShapeDtypeStruct((B, c2r.shape[1]), jnp.float32),
        scratch_shapes=[pltpu.VMEM((B, N), jnp.float32)],
    )(y, w1c, b1c, W2, b2c, M1, c1r, M2, c2r)


# speedup vs baseline: 1.5215x; 1.3933x over previous
import functools

import jax
import jax.numpy as jnp
from jax.experimental import pallas as pl
from jax.experimental.pallas import tpu as pltpu


def _gcn_body(y_ref, w1_ref, b1_ref, w2_ref, b2_ref, m1_ref, c1_ref,
              m2_ref, c2_ref, out_ref, r_scr, *, n_batch):
    b = pl.program_id(0)
    a = y_ref[0]
    n = a.shape[0]
    nh = w1_ref.shape[0]
    b1c = b1_ref[...].reshape(nh, 1)
    b2c = b2_ref[...].reshape(nh, 1)
    c1r = c1_ref[...].reshape(1, -1)
    c2r = c2_ref[...].reshape(1, -1)

    tile = 128
    row_i = jax.lax.broadcasted_iota(jnp.int32, (tile, tile), 0)
    col_i = jax.lax.broadcasted_iota(jnp.int32, (tile, tile), 1)
    mask = row_i == col_i
    diag = jnp.concatenate(
        [jnp.sum(jnp.where(mask,
                           y_ref[0, t * tile:(t + 1) * tile,
                                 t * tile:(t + 1) * tile], 0.0),
                 axis=0, keepdims=True)
         for t in range(n // tile)], axis=1)
    loop_w = jnp.where(diag == 0.0, 1.0, 0.0)
    deg = jnp.sum(a, axis=0, keepdims=True) + loop_w
    dinv = jnp.where(deg > 0.0, jax.lax.rsqrt(jnp.where(deg > 0.0, deg, 1.0)),
                     0.0)

    a_bf = a.astype(jnp.bfloat16)

    t1 = jnp.dot(dinv.astype(jnp.bfloat16), a_bf,
                 preferred_element_type=jnp.float32)
    s = dinv * (t1 + dinv * loop_w)
    x1t = jnp.maximum(w1_ref[...] * s + b1c, 0.0)

    h2t = jnp.dot(w2_ref[...], x1t,
                  preferred_element_type=jnp.float32)
    gt = dinv * h2t
    zt = jnp.dot(gt.astype(jnp.bfloat16), a_bf,
                 preferred_element_type=jnp.float32)
    out2t = dinv * (zt + loop_w * gt) + b2c
    r_scr[pl.ds(b, 1), :] = jnp.max(out2t, axis=0, keepdims=True)

    @pl.when(b == n_batch - 1)
    def _():
        rr = r_scr[...]
        o1 = jax.lax.dot_general(
            rr, m1_ref[...], (((1,), (1,)), ((), ())),
            preferred_element_type=jnp.float32) + c1r
        o2 = jax.lax.dot_general(
            o1, m2_ref[...], (((1,), (1,)), ((), ())),
            preferred_element_type=jnp.float32) + c2r
        out_ref[...] = o2


def kernel(y, W1, b1, W2, b2, M1, c1, M2, c2):
    B, N = y.shape[0], y.shape[1]

    vmem = pl.BlockSpec(memory_space=pltpu.MemorySpace.VMEM)
    return pl.pallas_call(
        functools.partial(_gcn_body, n_batch=B),
        grid=(B,),
        in_specs=[
            pl.BlockSpec((1, N, N), lambda b: (b, 0, 0)),
            vmem, vmem, vmem, vmem, vmem, vmem, vmem, vmem,
        ],
        out_specs=pl.BlockSpec((B, M2.shape[0]), lambda b: (0, 0)),
        out_shape=jax.ShapeDtypeStruct((B, M2.shape[0]), jnp.float32),
        scratch_shapes=[pltpu.VMEM((B, N), jnp.float32)],
    )(y, W1, b1, W2, b2, M1, c1, M2, c2)
